# baseline (speedup 1.0000x reference)
import functools

import jax
import jax.numpy as jnp
from jax import lax
from jax.experimental import pallas as pl
from jax.experimental.pallas import tpu as pltpu
from jax.experimental.pallas import tpu_sc as plsc

_N = 10000
_CH = 128
_HEADS = 4
_DH = 32
_EPS = 1e-5
_BQ = 400
_KCH = 128
_NTILES = 16
_ROWS_PER_TILE = _N // _NTILES


_VW = 40


def _enc_body(x_ref, pe_ref, peg, peb, new_ref, plw_ref, bcat_ref, c_ref,
              qkw_ref, qkb_ref, vxw_ref, vxb_ref, h_ref, y_ref, qk_ref, vx_ref):
    pe = pe_ref[...]
    m = jnp.mean(pe, axis=0, keepdims=True)
    v = jnp.mean((pe - m) ** 2, axis=0, keepdims=True)
    xpe = peg[...] * (pe - m) / jnp.sqrt(v + _EPS) + peb[...]
    h = (jnp.dot(x_ref[...], new_ref[...], preferred_element_type=jnp.float32)
         + jnp.dot(xpe, plw_ref[...], preferred_element_type=jnp.float32)
         + bcat_ref[...])
    h_ref[...] = h
    y_ref[...] = jnp.maximum(h + c_ref[...], 0.0)
    qk_ref[...] = (jnp.dot(h, qkw_ref[...], preferred_element_type=jnp.float32)
                   + qkb_ref[...]).astype(jnp.bfloat16)
    vx_ref[...] = (jnp.dot(h, vxw_ref[...], preferred_element_type=jnp.float32)
                   + vxb_ref[...]).astype(jnp.bfloat16)


def _encode(x, pe, peg, peb, new_pad, plw_pad, bcat, c, qkw, qkb, vxw, vxb):
    return pl.pallas_call(
        _enc_body,
        out_shape=[
            jax.ShapeDtypeStruct((_N, _CH), jnp.float32),
            jax.ShapeDtypeStruct((_N, _CH), jnp.float32),
            jax.ShapeDtypeStruct((_N, 2 * _CH), jnp.bfloat16),
            jax.ShapeDtypeStruct((_N, _HEADS * _VW), jnp.bfloat16),
        ],
    )(x, pe, peg, peb, new_pad, plw_pad, bcat, c, qkw, qkb, vxw, vxb)


def _make_segsum(n_edges):
    n_chunks = n_edges // _KCH
    base_chunks = n_chunks // 32
    rem = n_chunks - base_chunks * 32
    n_pairs = (base_chunks + (1 if rem else 0) + 1) // 2
    mesh = plsc.VectorSubcoreMesh(core_axis_name="c", subcore_axis_name="s")

    @functools.partial(
        pl.kernel,
        mesh=mesh,
        out_type=jax.ShapeDtypeStruct((2, _NTILES, _ROWS_PER_TILE, _CH), jnp.float32),
        scratch_types=[
            pltpu.VMEM((_KCH,), jnp.int32),
            pltpu.VMEM((_KCH,), jnp.int32),
            pltpu.VMEM((_KCH, _CH), jnp.float32),
            pltpu.VMEM((_KCH,), jnp.int32),
            pltpu.VMEM((_KCH,), jnp.int32),
            pltpu.VMEM((_KCH, _CH), jnp.float32),
            pltpu.VMEM_SHARED((_N, _CH), jnp.float32),
            pltpu.SemaphoreType.DMA,
            pltpu.SemaphoreType.DMA,
        ],
    )
    def segsum(y_hbm, src_hbm, dst_hbm, zeros_hbm, out_hbm,
               sidx0, didx0, rows0, sidx1, didx1, rows1, acc_sh, sem0, sem1):
        c = lax.axis_index("c")
        s = lax.axis_index("s")
        pltpu.sync_copy(zeros_hbm, acc_sh.at[pl.ds(s * _ROWS_PER_TILE, _ROWS_PER_TILE)])
        plsc.subcore_barrier()

        wid = c * _NTILES + s
        nloc = base_chunks + jnp.where(wid < rem, 1, 0)

        def idx_copy(i, sidx, didx):
            base = (wid + 32 * i) * _KCH
            pltpu.sync_copy(src_hbm.at[pl.ds(base, _KCH)], sidx)
            pltpu.sync_copy(dst_hbm.at[pl.ds(base, _KCH)], didx)

        idx_copy(0, sidx0, didx0)
        pltpu.async_copy(y_hbm.at[sidx0], rows0, sem0)

        def body(j, carry):
            i1 = 2 * j + 1

            @pl.when(i1 < nloc)
            def _():
                idx_copy(i1, sidx1, didx1)
                pltpu.async_copy(y_hbm.at[sidx1], rows1, sem1)

            @pl.when(2 * j < nloc)
            def _():
                pltpu.make_async_copy(y_hbm.at[sidx0], rows0, sem0).wait()
                pltpu.sync_copy(rows0, acc_sh.at[didx0], add=True)

            @pl.when(2 * j + 2 < nloc)
            def _():
                idx_copy(2 * j + 2, sidx0, didx0)
                pltpu.async_copy(y_hbm.at[sidx0], rows0, sem0)

            @pl.when(i1 < nloc)
            def _():
                pltpu.make_async_copy(y_hbm.at[sidx1], rows1, sem1).wait()
                pltpu.sync_copy(rows1, acc_sh.at[didx1], add=True)

            return carry

        lax.fori_loop(0, n_pairs, body, 0)
        plsc.subcore_barrier()
        pltpu.sync_copy(acc_sh.at[pl.ds(s * _ROWS_PER_TILE, _ROWS_PER_TILE)],
                        out_hbm.at[c, s])

    return segsum


def _segsum(y, src, dst):
    zeros = jnp.zeros((_ROWS_PER_TILE, _CH), jnp.float32)
    out = _make_segsum(src.shape[0])(y, src, dst, zeros)
    return out.reshape(2, _N, _CH)


def _attn_body(qk_blk_ref, qk_all_ref, vx_all_ref, h_blk_ref, outw_ref,
               outb_ref, o_ref):
    acc = h_blk_ref[...] + outb_ref[...]
    for hd in range(_HEADS):
        qh = qk_blk_ref[:, hd * _DH:(hd + 1) * _DH]
        kh = qk_all_ref[:, _CH + hd * _DH:_CH + (hd + 1) * _DH]
        vh = vx_all_ref[:, hd * _VW:(hd + 1) * _VW]
        s = lax.dot_general(qh, kh, (((1,), (1,)), ((), ())),
                            preferred_element_type=jnp.float32)
        p = jnp.exp(s).astype(jnp.bfloat16)
        o_ext = jnp.dot(p, vh, preferred_element_type=jnp.float32)
        oh = o_ext[:, :_DH] / o_ext[:, _DH:_DH + 1]
        acc = acc + jnp.dot(oh, outw_ref[hd * _DH:(hd + 1) * _DH, :],
                            preferred_element_type=jnp.float32)
    o_ref[...] = acc


def _attention(qk, vx, h, outw, outb):
    nblk = _N // _BQ
    return pl.pallas_call(
        _attn_body,
        grid=(nblk,),
        in_specs=[
            pl.BlockSpec((_BQ, 2 * _CH), lambda i: (i, 0)),
            pl.BlockSpec((_N, 2 * _CH), lambda i: (0, 0)),
            pl.BlockSpec((_N, _HEADS * _VW), lambda i: (0, 0)),
            pl.BlockSpec((_BQ, _CH), lambda i: (i, 0)),
            pl.BlockSpec((_CH, _CH), lambda i: (0, 0)),
            pl.BlockSpec((1, _CH), lambda i: (0, 0)),
        ],
        out_specs=pl.BlockSpec((_BQ, _CH), lambda i: (i, 0)),
        out_shape=jax.ShapeDtypeStruct((_N, _CH), jnp.float32),
    )(qk, qk, vx, h, outw, outb)


def _bn(t, g, b):
    m = jnp.mean(t, axis=0, keepdims=True)
    v = jnp.mean((t - m) ** 2, axis=0, keepdims=True)
    return g * (t - m) / jnp.sqrt(v + _EPS) + b


def _final_body(h_ref, agg_ref, h2pre_ref, w1_ref, b1_ref, w2_ref, b2_ref,
                n1g, n1b, n2g, n2b, n3g, n3b,
                f1w, f1b, f2w, f2b, m1w, m1b, m2w, m2b, m3w, m3b, o_ref):
    h = h_ref[...]
    g = h + agg_ref[0] + agg_ref[1]
    gm = jnp.maximum(jnp.dot(g, w1_ref[...], preferred_element_type=jnp.float32)
                     + b1_ref[...], 0.0)
    gine_pre = (jnp.dot(gm, w2_ref[...], preferred_element_type=jnp.float32)
                + b2_ref[...] + h)
    out = _bn(gine_pre, n1g[...], n1b[...]) + _bn(h2pre_ref[...], n2g[...], n2b[...])
    ffm = jnp.maximum(jnp.dot(out, f1w[...], preferred_element_type=jnp.float32)
                      + f1b[...], 0.0)
    out2 = out + jnp.dot(ffm, f2w[...], preferred_element_type=jnp.float32) + f2b[...]
    out3 = _bn(out2, n3g[...], n3b[...])
    t = jnp.maximum(jnp.dot(out3, m1w[...], preferred_element_type=jnp.float32)
                    + m1b[...], 0.0)
    t = jnp.maximum(jnp.dot(t, m2w[...], preferred_element_type=jnp.float32)
                    + m2b[...], 0.0)
    o_ref[...] = jnp.dot(t, m3w[...], preferred_element_type=jnp.float32) + m3b[...]


def _final(h, agg, h2pre, args):
    return pl.pallas_call(
        _final_body,
        out_shape=jax.ShapeDtypeStruct((_N, 2), jnp.float32),
    )(h, agg, h2pre, *args)


def _row(v):
    return v.reshape(1, -1)


def kernel(x, pe, edge_index, batch, params):
    p = params
    lp = p['layers'][0]
    wl = pe.shape[1]

    new_pad = jnp.zeros((x.shape[1], _CH), jnp.float32).at[:, :_CH - 8].set(p['ne_w'].T)
    plw_pad = jnp.zeros((wl, _CH), jnp.float32).at[:, _CH - 8:].set(p['pl_w'].T)
    bcat = _row(jnp.concatenate([p['ne_b'], p['pl_b']]))
    c = _row(p['ee_w'][:, 0] + p['ee_b'])

    inwt = lp['in_w'].T
    inb = lp['in_b']
    col_scale = jnp.full((_CH,), _DH ** -0.5, jnp.float32)
    qkw = jnp.concatenate([inwt[:, :_CH] * col_scale[None, :],
                           inwt[:, _CH:2 * _CH]], axis=1)
    qkb = jnp.concatenate([inb[:_CH] * col_scale, inb[_CH:2 * _CH]])
    vxw = jnp.zeros((_CH, _HEADS * _VW), jnp.float32)
    vxb = jnp.zeros((_HEADS * _VW,), jnp.float32)
    for hd in range(_HEADS):
        vxw = vxw.at[:, hd * _VW:hd * _VW + _DH].set(
            inwt[:, 2 * _CH + hd * _DH:2 * _CH + (hd + 1) * _DH])
        vxb = vxb.at[hd * _VW:hd * _VW + _DH].set(
            inb[2 * _CH + hd * _DH:2 * _CH + (hd + 1) * _DH])
        vxb = vxb.at[hd * _VW + _DH].set(1.0)

    h, y, qk, vx = _encode(x, pe, _row(p['pe_ng']), _row(p['pe_nb']),
                           new_pad, plw_pad, bcat, c,
                           qkw, _row(qkb), vxw, _row(vxb))

    agg = _segsum(y, edge_index[0], edge_index[1])
    h2pre = _attention(qk, vx, h, lp['out_w'].T, _row(lp['out_b']))

    args = (lp['nn_w1'].T, _row(lp['nn_b1']), lp['nn_w2'].T, _row(lp['nn_b2']),
            _row(lp['n1g']), _row(lp['n1b']), _row(lp['n2g']), _row(lp['n2b']),
            _row(lp['n3g']), _row(lp['n3b']),
            lp['ff_w1'].T, _row(lp['ff_b1']), lp['ff_w2'].T, _row(lp['ff_b2']),
            p['m_w1'].T, _row(p['m_b1']), p['m_w2'].T, _row(p['m_b2']),
            p['m_w3'].T, _row(p['m_b3']))
    return _final(h, agg, h2pre, args)


# speedup vs baseline: 1.4030x; 1.4030x over previous
import functools

import jax
import jax.numpy as jnp
from jax import lax
from jax.experimental import pallas as pl
from jax.experimental.pallas import tpu as pltpu
from jax.experimental.pallas import tpu_sc as plsc

_N = 10000
_CH = 128
_HEADS = 4
_DH = 32
_EPS = 1e-5
_BQ = 400
_KCH = 128
_NTILES = 16
_ROWS_PER_TILE = _N // _NTILES


def _enc_body(x_ref, pe_ref, peg, peb, new_ref, plw_ref, bcat_ref, c_ref,
              qkw_ref, qkb_ref, vow_ref, vob_ref, h_ref, y_ref, qkv_ref, vo_ref):
    pe = pe_ref[...]
    m = jnp.mean(pe, axis=0, keepdims=True)
    v = jnp.mean((pe - m) ** 2, axis=0, keepdims=True)
    xpe = peg[...] * (pe - m) / jnp.sqrt(v + _EPS) + peb[...]
    h = (jnp.dot(x_ref[...], new_ref[...], preferred_element_type=jnp.float32)
         + jnp.dot(xpe, plw_ref[...], preferred_element_type=jnp.float32)
         + bcat_ref[...])
    h_ref[...] = h
    y_ref[...] = jnp.maximum(h + c_ref[...], 0.0)
    qkv_ref[...] = (jnp.dot(h, qkw_ref[...], preferred_element_type=jnp.float32)
                    + qkb_ref[...]).astype(jnp.bfloat16)
    vo_ref[...] = (jnp.dot(h, vow_ref[...], preferred_element_type=jnp.float32)
                   + vob_ref[...]).astype(jnp.bfloat16)


def _encode(x, pe, peg, peb, new_pad, plw_pad, bcat, c, qkw, qkb, vow, vob):
    return pl.pallas_call(
        _enc_body,
        out_shape=[
            jax.ShapeDtypeStruct((_N, _CH), jnp.float32),
            jax.ShapeDtypeStruct((_N, _CH), jnp.float32),
            jax.ShapeDtypeStruct((_N, 2 * _CH), jnp.bfloat16),
            jax.ShapeDtypeStruct((_N, _HEADS * _CH), jnp.bfloat16),
        ],
    )(x, pe, peg, peb, new_pad, plw_pad, bcat, c, qkw, qkb, vow, vob)


def _make_segsum(n_edges):
    n_chunks = n_edges // _KCH
    base_chunks = n_chunks // 32
    rem = n_chunks - base_chunks * 32
    n_pairs = (base_chunks + (1 if rem else 0) + 1) // 2
    mesh = plsc.VectorSubcoreMesh(core_axis_name="c", subcore_axis_name="s")

    @functools.partial(
        pl.kernel,
        mesh=mesh,
        out_type=jax.ShapeDtypeStruct((2, _NTILES, _ROWS_PER_TILE, _CH), jnp.float32),
        scratch_types=[
            pltpu.VMEM((_KCH,), jnp.int32),
            pltpu.VMEM((_KCH,), jnp.int32),
            pltpu.VMEM((_KCH, _CH), jnp.float32),
            pltpu.VMEM((_KCH,), jnp.int32),
            pltpu.VMEM((_KCH,), jnp.int32),
            pltpu.VMEM((_KCH, _CH), jnp.float32),
            pltpu.VMEM_SHARED((_N, _CH), jnp.float32),
            pltpu.SemaphoreType.DMA,
            pltpu.SemaphoreType.DMA,
        ],
    )
    def segsum(y_hbm, src_hbm, dst_hbm, zeros_hbm, out_hbm,
               sidx0, didx0, rows0, sidx1, didx1, rows1, acc_sh, sem0, sem1):
        c = lax.axis_index("c")
        s = lax.axis_index("s")
        pltpu.sync_copy(zeros_hbm, acc_sh.at[pl.ds(s * _ROWS_PER_TILE, _ROWS_PER_TILE)])
        plsc.subcore_barrier()

        wid = c * _NTILES + s
        nloc = base_chunks + jnp.where(wid < rem, 1, 0)

        def idx_copy(i, sidx, didx):
            base = (wid + 32 * i) * _KCH
            pltpu.sync_copy(src_hbm.at[pl.ds(base, _KCH)], sidx)
            pltpu.sync_copy(dst_hbm.at[pl.ds(base, _KCH)], didx)

        idx_copy(0, sidx0, didx0)
        pltpu.async_copy(y_hbm.at[sidx0], rows0, sem0)

        def body(j, carry):
            i1 = 2 * j + 1

            @pl.when(i1 < nloc)
            def _():
                idx_copy(i1, sidx1, didx1)
                pltpu.async_copy(y_hbm.at[sidx1], rows1, sem1)

            @pl.when(2 * j < nloc)
            def _():
                pltpu.make_async_copy(y_hbm.at[sidx0], rows0, sem0).wait()
                pltpu.sync_copy(rows0, acc_sh.at[didx0], add=True)

            @pl.when(2 * j + 2 < nloc)
            def _():
                idx_copy(2 * j + 2, sidx0, didx0)
                pltpu.async_copy(y_hbm.at[sidx0], rows0, sem0)

            @pl.when(i1 < nloc)
            def _():
                pltpu.make_async_copy(y_hbm.at[sidx1], rows1, sem1).wait()
                pltpu.sync_copy(rows1, acc_sh.at[didx1], add=True)

            return carry

        lax.fori_loop(0, n_pairs, body, 0)
        plsc.subcore_barrier()
        pltpu.sync_copy(acc_sh.at[pl.ds(s * _ROWS_PER_TILE, _ROWS_PER_TILE)],
                        out_hbm.at[c, s])

    return segsum


def _segsum(y, src, dst):
    zeros = jnp.zeros((_ROWS_PER_TILE, _CH), jnp.float32)
    out = _make_segsum(src.shape[0])(y, src, dst, zeros)
    return out.reshape(2, _N, _CH)


def _attn_body(qk_blk_ref, qk_all_ref, vo_all_ref, h_blk_ref, outb_ref, o_ref):
    def qkmm(hd):
        return lax.dot_general(qk_blk_ref[:, hd * _DH:(hd + 1) * _DH],
                               qk_all_ref[:, _CH + hd * _DH:_CH + (hd + 1) * _DH],
                               (((1,), (1,)), ((), ())),
                               preferred_element_type=jnp.float32)

    def softmax_num(s):
        pf = jnp.exp(s)
        return pf.astype(jnp.bfloat16), jnp.sum(pf, axis=1, keepdims=True)

    def avmm(pb, hd):
        return jnp.dot(pb, vo_all_ref[:, hd * _CH:(hd + 1) * _CH],
                       preferred_element_type=jnp.float32)

    acc = h_blk_ref[...] + outb_ref[...]
    s_cur = qkmm(0)
    for hd in range(_HEADS):
        pb, dn = softmax_num(s_cur)
        if hd + 1 < _HEADS:
            s_cur = qkmm(hd + 1)
        acc = acc + avmm(pb, hd) / dn
    o_ref[...] = acc


def _attention(qk, vo, h, outb):
    nblk = _N // _BQ
    return pl.pallas_call(
        _attn_body,
        grid=(nblk,),
        in_specs=[
            pl.BlockSpec((_BQ, 2 * _CH), lambda i: (i, 0)),
            pl.BlockSpec((_N, 2 * _CH), lambda i: (0, 0)),
            pl.BlockSpec((_N, _HEADS * _CH), lambda i: (0, 0)),
            pl.BlockSpec((_BQ, _CH), lambda i: (i, 0)),
            pl.BlockSpec((1, _CH), lambda i: (0, 0)),
        ],
        out_specs=pl.BlockSpec((_BQ, _CH), lambda i: (i, 0)),
        out_shape=jax.ShapeDtypeStruct((_N, _CH), jnp.float32),
    )(qk, qk, vo, h, outb)


def _bn(t, g, b):
    m = jnp.mean(t, axis=0, keepdims=True)
    v = jnp.mean((t - m) ** 2, axis=0, keepdims=True)
    return g * (t - m) / jnp.sqrt(v + _EPS) + b


def _final_body(h_ref, agg_ref, h2pre_ref, w1_ref, b1_ref, w2_ref, b2_ref,
                n1g, n1b, n2g, n2b, n3g, n3b,
                f1w, f1b, f2w, f2b, m1w, m1b, m2w, m2b, m3w, m3b, o_ref):
    h = h_ref[...]
    g = h + agg_ref[0] + agg_ref[1]
    gm = jnp.maximum(jnp.dot(g, w1_ref[...], preferred_element_type=jnp.float32)
                     + b1_ref[...], 0.0)
    gine_pre = (jnp.dot(gm, w2_ref[...], preferred_element_type=jnp.float32)
                + b2_ref[...] + h)
    out = _bn(gine_pre, n1g[...], n1b[...]) + _bn(h2pre_ref[...], n2g[...], n2b[...])
    ffm = jnp.maximum(jnp.dot(out, f1w[...], preferred_element_type=jnp.float32)
                      + f1b[...], 0.0)
    out2 = out + jnp.dot(ffm, f2w[...], preferred_element_type=jnp.float32) + f2b[...]
    out3 = _bn(out2, n3g[...], n3b[...])
    t = jnp.maximum(jnp.dot(out3, m1w[...], preferred_element_type=jnp.float32)
                    + m1b[...], 0.0)
    t = jnp.maximum(jnp.dot(t, m2w[...], preferred_element_type=jnp.float32)
                    + m2b[...], 0.0)
    o_ref[...] = jnp.dot(t, m3w[...], preferred_element_type=jnp.float32) + m3b[...]


def _final(h, agg, h2pre, args):
    return pl.pallas_call(
        _final_body,
        out_shape=jax.ShapeDtypeStruct((_N, 2), jnp.float32),
    )(h, agg, h2pre, *args)


def _row(v):
    return v.reshape(1, -1)


def kernel(x, pe, edge_index, batch, params):
    p = params
    lp = p['layers'][0]
    wl = pe.shape[1]

    new_pad = jnp.zeros((x.shape[1], _CH), jnp.float32).at[:, :_CH - 8].set(p['ne_w'].T)
    plw_pad = jnp.zeros((wl, _CH), jnp.float32).at[:, _CH - 8:].set(p['pl_w'].T)
    bcat = _row(jnp.concatenate([p['ne_b'], p['pl_b']]))
    c = _row(p['ee_w'][:, 0] + p['ee_b'])

    inwt = lp['in_w'].T
    inb = lp['in_b']
    outwt = lp['out_w'].T
    scale = jnp.full((_CH,), _DH ** -0.5, jnp.float32)
    qkw = jnp.concatenate([inwt[:, :_CH] * scale[None, :],
                           inwt[:, _CH:2 * _CH]], axis=1)
    qkb = jnp.concatenate([inb[:_CH] * scale, inb[_CH:2 * _CH]])
    vo_w = []
    vo_b = []
    for hd in range(_HEADS):
        sl = slice(2 * _CH + hd * _DH, 2 * _CH + (hd + 1) * _DH)
        ow = outwt[hd * _DH:(hd + 1) * _DH, :]
        vo_w.append(inwt[:, sl] @ ow)
        vo_b.append(inb[sl] @ ow)
    vow = jnp.concatenate(vo_w, axis=1)
    vob = jnp.concatenate(vo_b)

    h, y, qk, vo = _encode(x, pe, _row(p['pe_ng']), _row(p['pe_nb']),
                           new_pad, plw_pad, bcat, c,
                           qkw, _row(qkb), vow, _row(vob))

    agg = _segsum(y, edge_index[0], edge_index[1])
    h2pre = _attention(qk, vo, h, _row(lp['out_b']))

    args = (lp['nn_w1'].T, _row(lp['nn_b1']), lp['nn_w2'].T, _row(lp['nn_b2']),
            _row(lp['n1g']), _row(lp['n1b']), _row(lp['n2g']), _row(lp['n2b']),
            _row(lp['n3g']), _row(lp['n3b']),
            lp['ff_w1'].T, _row(lp['ff_b1']), lp['ff_w2'].T, _row(lp['ff_b2']),
            p['m_w1'].T, _row(p['m_b1']), p['m_w2'].T, _row(p['m_b2']),
            p['m_w3'].T, _row(p['m_b3']))
    return _final(h, agg, h2pre, args)


# speedup vs baseline: 1.4451x; 1.0301x over previous
import functools

import jax
import jax.numpy as jnp
from jax import lax
from jax.experimental import pallas as pl
from jax.experimental.pallas import tpu as pltpu
from jax.experimental.pallas import tpu_sc as plsc

_N = 10000
_CH = 128
_HEADS = 4
_DH = 32
_EPS = 1e-5
_BQ = 400
_KCH = 128
_NTILES = 16
_ROWS_PER_TILE = _N // _NTILES

_C11 = (((1,), (1,)), ((), ()))


def _enc_body(x_ref, pe_ref, peg, peb, new_ref, neb_ref, plw_ref, plb_ref,
              c_ref, inw_ref, inb_ref, outw_ref,
              h_ref, y_ref, qk_ref, vo_ref):
    pe = pe_ref[...]
    m = jnp.mean(pe, axis=0, keepdims=True)
    v = jnp.mean((pe - m) ** 2, axis=0, keepdims=True)
    xpe = peg[...] * (pe - m) / jnp.sqrt(v + _EPS) + peb[...]

    h = (lax.dot_general(x_ref[...], new_ref[...], _C11,
                         preferred_element_type=jnp.float32)
         + lax.dot_general(xpe, plw_ref[...], _C11,
                           preferred_element_type=jnp.float32)
         + neb_ref[...] + plb_ref[...])
    h_ref[...] = h
    y_ref[...] = jnp.maximum(h + c_ref[...], 0.0)

    inw = inw_ref[...]
    inb = inb_ref[...]
    outw = outw_ref[...]
    qk = (lax.dot_general(h, inw[:2 * _CH, :], _C11,
                          preferred_element_type=jnp.float32)
          + inb[:, :2 * _CH])
    scale = jnp.where(lax.broadcasted_iota(jnp.int32, (1, 2 * _CH), 1) < _CH,
                      _DH ** -0.5, 1.0)
    qk_ref[...] = (qk * scale).astype(jnp.bfloat16)

    ms = []
    bs = []
    for hd in range(_HEADS):
        wv = inw[2 * _CH + hd * _DH:2 * _CH + (hd + 1) * _DH, :]
        ow = outw[:, hd * _DH:(hd + 1) * _DH]
        ms.append(lax.dot_general(wv, ow, (((0,), (1,)), ((), ())),
                                  preferred_element_type=jnp.float32))
        bs.append(lax.dot_general(inb[:, 2 * _CH + hd * _DH:
                                      2 * _CH + (hd + 1) * _DH], ow, _C11,
                                  preferred_element_type=jnp.float32))
    mcat = jnp.concatenate(ms, axis=1)
    bscat = jnp.concatenate(bs, axis=1)
    vo_ref[...] = (jnp.dot(h, mcat, preferred_element_type=jnp.float32)
                   + bscat).astype(jnp.bfloat16)


def _encode(x, pe, peg, peb, new_pad, neb_pad, plw_pad, plb_pad, c, inw, inb,
            outw):
    return pl.pallas_call(
        _enc_body,
        out_shape=[
            jax.ShapeDtypeStruct((_N, _CH), jnp.float32),
            jax.ShapeDtypeStruct((_N, _CH), jnp.float32),
            jax.ShapeDtypeStruct((_N, 2 * _CH), jnp.bfloat16),
            jax.ShapeDtypeStruct((_N, _HEADS * _CH), jnp.bfloat16),
        ],
    )(x, pe, peg, peb, new_pad, neb_pad, plw_pad, plb_pad, c, inw, inb, outw)


def _make_segsum(n_edges):
    n_chunks = n_edges // _KCH
    base_chunks = n_chunks // 32
    rem = n_chunks - base_chunks * 32
    n_pairs = (base_chunks + (1 if rem else 0) + 1) // 2
    mesh = plsc.VectorSubcoreMesh(core_axis_name="c", subcore_axis_name="s")

    @functools.partial(
        pl.kernel,
        mesh=mesh,
        out_type=jax.ShapeDtypeStruct((2, _NTILES, _ROWS_PER_TILE, _CH), jnp.float32),
        scratch_types=[
            pltpu.VMEM((_KCH,), jnp.int32),
            pltpu.VMEM((_KCH,), jnp.int32),
            pltpu.VMEM((_KCH, _CH), jnp.float32),
            pltpu.VMEM((_KCH,), jnp.int32),
            pltpu.VMEM((_KCH,), jnp.int32),
            pltpu.VMEM((_KCH, _CH), jnp.float32),
            pltpu.VMEM_SHARED((_N, _CH), jnp.float32),
            pltpu.SemaphoreType.DMA,
            pltpu.SemaphoreType.DMA,
        ],
    )
    def segsum(y_hbm, src_hbm, dst_hbm, zeros_hbm, out_hbm,
               sidx0, didx0, rows0, sidx1, didx1, rows1, acc_sh, sem0, sem1):
        c = lax.axis_index("c")
        s = lax.axis_index("s")
        pltpu.sync_copy(zeros_hbm, acc_sh.at[pl.ds(s * _ROWS_PER_TILE, _ROWS_PER_TILE)])
        plsc.subcore_barrier()

        wid = c * _NTILES + s
        nloc = base_chunks + jnp.where(wid < rem, 1, 0)

        def idx_copy(i, sidx, didx):
            base = (wid + 32 * i) * _KCH
            pltpu.sync_copy(src_hbm.at[pl.ds(base, _KCH)], sidx)
            pltpu.sync_copy(dst_hbm.at[pl.ds(base, _KCH)], didx)

        idx_copy(0, sidx0, didx0)
        pltpu.async_copy(y_hbm.at[sidx0], rows0, sem0)

        def body(j, carry):
            i1 = 2 * j + 1

            @pl.when(i1 < nloc)
            def _():
                idx_copy(i1, sidx1, didx1)
                pltpu.async_copy(y_hbm.at[sidx1], rows1, sem1)

            @pl.when(2 * j < nloc)
            def _():
                pltpu.make_async_copy(y_hbm.at[sidx0], rows0, sem0).wait()
                pltpu.sync_copy(rows0, acc_sh.at[didx0], add=True)

            @pl.when(2 * j + 2 < nloc)
            def _():
                idx_copy(2 * j + 2, sidx0, didx0)
                pltpu.async_copy(y_hbm.at[sidx0], rows0, sem0)

            @pl.when(i1 < nloc)
            def _():
                pltpu.make_async_copy(y_hbm.at[sidx1], rows1, sem1).wait()
                pltpu.sync_copy(rows1, acc_sh.at[didx1], add=True)

            return carry

        lax.fori_loop(0, n_pairs, body, 0)
        plsc.subcore_barrier()
        pltpu.sync_copy(acc_sh.at[pl.ds(s * _ROWS_PER_TILE, _ROWS_PER_TILE)],
                        out_hbm.at[c, s])

    return segsum


def _segsum(y, src, dst):
    zeros = jnp.zeros((_ROWS_PER_TILE, _CH), jnp.float32)
    out = _make_segsum(src.shape[0])(y, src, dst, zeros)
    return out.reshape(2, _N, _CH)


def _attn_body(qk_blk_ref, qk_all_ref, vo_all_ref, h_blk_ref, outb_ref, o_ref):
    def qkmm(hd):
        return lax.dot_general(qk_blk_ref[:, hd * _DH:(hd + 1) * _DH],
                               qk_all_ref[:, _CH + hd * _DH:_CH + (hd + 1) * _DH],
                               _C11, preferred_element_type=jnp.float32)

    def softmax_num(s):
        pf = jnp.exp(s)
        return pf.astype(jnp.bfloat16), jnp.sum(pf, axis=1, keepdims=True)

    def avmm(pb, hd):
        return jnp.dot(pb, vo_all_ref[:, hd * _CH:(hd + 1) * _CH],
                       preferred_element_type=jnp.float32)

    acc = h_blk_ref[...] + outb_ref[...]
    s_cur = qkmm(0)
    for hd in range(_HEADS):
        pb, dn = softmax_num(s_cur)
        if hd + 1 < _HEADS:
            s_cur = qkmm(hd + 1)
        acc = acc + avmm(pb, hd) / dn
    o_ref[...] = acc


def _attention(qk, vo, h, outb):
    nblk = _N // _BQ
    return pl.pallas_call(
        _attn_body,
        grid=(nblk,),
        in_specs=[
            pl.BlockSpec((_BQ, 2 * _CH), lambda i: (i, 0)),
            pl.BlockSpec((_N, 2 * _CH), lambda i: (0, 0)),
            pl.BlockSpec((_N, _HEADS * _CH), lambda i: (0, 0)),
            pl.BlockSpec((_BQ, _CH), lambda i: (i, 0)),
            pl.BlockSpec((1, _CH), lambda i: (0, 0)),
        ],
        out_specs=pl.BlockSpec((_BQ, _CH), lambda i: (i, 0)),
        out_shape=jax.ShapeDtypeStruct((_N, _CH), jnp.float32),
    )(qk, qk, vo, h, outb)


def _bn(t, g, b):
    m = jnp.mean(t, axis=0, keepdims=True)
    v = jnp.mean((t - m) ** 2, axis=0, keepdims=True)
    return g * (t - m) / jnp.sqrt(v + _EPS) + b


def _final_body(h_ref, agg_ref, h2pre_ref, w1_ref, b1_ref, w2_ref, b2_ref,
                n1g, n1b, n2g, n2b, n3g, n3b,
                f1w, f1b, f2w, f2b, m1w, m1b, m2w, m2b, m3w, m3b, o_ref):
    h = h_ref[...]
    g = h + agg_ref[0] + agg_ref[1]
    gm = jnp.maximum(lax.dot_general(g, w1_ref[...], _C11,
                                     preferred_element_type=jnp.float32)
                     + b1_ref[...], 0.0)
    gine_pre = (lax.dot_general(gm, w2_ref[...], _C11,
                                preferred_element_type=jnp.float32)
                + b2_ref[...] + h)
    out = _bn(gine_pre, n1g[...], n1b[...]) + _bn(h2pre_ref[...], n2g[...], n2b[...])
    ffm = jnp.maximum(lax.dot_general(out, f1w[...], _C11,
                                      preferred_element_type=jnp.float32)
                      + f1b[...], 0.0)
    out2 = out + lax.dot_general(ffm, f2w[...], _C11,
                                 preferred_element_type=jnp.float32) + f2b[...]
    out3 = _bn(out2, n3g[...], n3b[...])
    t = jnp.maximum(lax.dot_general(out3, m1w[...], _C11,
                                    preferred_element_type=jnp.float32)
                    + m1b[...], 0.0)
    t = jnp.maximum(lax.dot_general(t, m2w[...], _C11,
                                    preferred_element_type=jnp.float32)
                    + m2b[...], 0.0)
    o_ref[...] = (lax.dot_general(t, m3w[...], _C11,
                                  preferred_element_type=jnp.float32)
                  + m3b[...])


def _final(h, agg, h2pre, args):
    return pl.pallas_call(
        _final_body,
        out_shape=jax.ShapeDtypeStruct((_N, 2), jnp.float32),
    )(h, agg, h2pre, *args)


def _row(v):
    return v.reshape(1, -1)


def kernel(x, pe, edge_index, batch, params):
    p = params
    lp = p['layers'][0]

    c = _row(p['ee_w'][:, 0] + p['ee_b'])

    neb_pad = _row(jnp.pad(p['ne_b'], (0, 8)))
    plb_pad = _row(jnp.pad(p['pl_b'], (_CH - 8, 0)))
    new_pad = jnp.pad(p['ne_w'], ((0, 8), (0, 0)))
    plw_pad = jnp.pad(p['pl_w'], ((_CH - 8, 0), (0, 0)))

    h, y, qk, vo = _encode(x, pe, _row(p['pe_ng']), _row(p['pe_nb']),
                           new_pad, neb_pad, plw_pad, plb_pad,
                           c, lp['in_w'], _row(lp['in_b']), lp['out_w'])

    agg = _segsum(y, edge_index[0], edge_index[1])
    h2pre = _attention(qk, vo, h, _row(lp['out_b']))

    args = (lp['nn_w1'], _row(lp['nn_b1']), lp['nn_w2'], _row(lp['nn_b2']),
            _row(lp['n1g']), _row(lp['n1b']), _row(lp['n2g']), _row(lp['n2b']),
            _row(lp['n3g']), _row(lp['n3b']),
            lp['ff_w1'], _row(lp['ff_b1']), lp['ff_w2'], _row(lp['ff_b2']),
            p['m_w1'], _row(p['m_b1']), p['m_w2'], _row(p['m_b2']),
            p['m_w3'], _row(p['m_b3']))
    return _final(h, agg, h2pre, args)
